# baseline (speedup 1.0000x reference)
import functools

import jax
import jax.numpy as jnp
from jax import lax
from jax.experimental import pallas as pl
from jax.experimental.pallas import tpu as pltpu
from jax.experimental.pallas import tpu_sc as plsc

_K = 1024
_D = 256
_ROWS = 8192
_BLK = 1024

_NC = 2
_NS = 16
_NW = _NC * _NS
_ROWS_PER_W = _ROWS // _NW


def _dist_argmin_body(z_ref, emb_ref, en_ref, idx_ref, minv_ref):
    zb = z_ref[0]
    emb = emb_ref[...]
    en = en_ref[...]
    rn = jnp.sum(zb * zb, axis=0, keepdims=True)
    p = lax.dot_general(emb, zb, (((1,), (0,)), ((), ())),
                        preferred_element_type=jnp.float32)
    d = (rn + en) - 2.0 * p
    minv = jnp.min(d, axis=0, keepdims=True)
    iota = lax.broadcasted_iota(jnp.int32, d.shape, 0)
    idx = jnp.min(jnp.where(d == minv, iota, jnp.int32(_K)),
                  axis=0, keepdims=True)
    idx_ref[...] = idx[None]
    minv_ref[...] = minv[None]


_S = 1024

_dist_argmin = pl.pallas_call(
    _dist_argmin_body,
    grid=(8,),
    in_specs=[
        pl.BlockSpec((1, _D, _S), lambda i: (i, 0, 0)),
        pl.BlockSpec((_K, _D), lambda i: (0, 0)),
        pl.BlockSpec((_K, 1), lambda i: (0, 0)),
    ],
    out_specs=[
        pl.BlockSpec((1, 1, _S), lambda i: (i, 0, 0)),
        pl.BlockSpec((1, 1, _S), lambda i: (i, 0, 0)),
    ],
    out_shape=[
        jax.ShapeDtypeStruct((8, 1, _S), jnp.int32),
        jax.ShapeDtypeStruct((8, 1, _S), jnp.float32),
    ],
)


_CB = 64
_CH = 32


@functools.lru_cache(maxsize=1)
def _make_sc_gather():
    @functools.partial(
        pl.kernel,
        mesh=plsc.VectorSubcoreMesh(core_axis_name="c", subcore_axis_name="s"),
        compiler_params=pltpu.CompilerParams(needs_layout_passes=False),
        out_type=jax.ShapeDtypeStruct((8, _D, _S), jnp.float32),
        scratch_types=[
            pltpu.VMEM((_CB * _S,), jnp.float32),
            pltpu.VMEM((_S,), jnp.int32),
            pltpu.VMEM((_CH, _S), jnp.float32),
        ],
    )
    def _sc_gather(tabt_hbm, idx_hbm, out_hbm, slab_v, idx_v, out_v):
        wid = lax.axis_index("s") * _NC + lax.axis_index("c")
        b = wid // 4
        c0 = (wid % 4) * _CB
        pltpu.sync_copy(tabt_hbm.at[pl.ds(c0 * _S, _CB * _S)], slab_v)
        pltpu.sync_copy(idx_hbm.at[b], idx_v)
        for h in range(_CB // _CH):
            cbase = h * _CH

            def body_i(i, carry, cbase=cbase):
                s0 = i * 16
                fidx = idx_v[pl.ds(s0, 16)] + jnp.int32(cbase * _S)
                for c in range(_CH):
                    out_v[c, pl.ds(s0, 16)] = plsc.load_gather(slab_v, [fidx])
                    if c + 1 < _CH:
                        fidx = fidx + jnp.int32(_S)
                return carry

            lax.fori_loop(0, _S // 16, body_i, 0)
            pltpu.sync_copy(out_v, out_hbm.at[b, pl.ds(c0 + cbase, _CH), :])

    return _sc_gather


def kernel(z, embedding):
    beta = 0.25
    B, C, H, W = z.shape
    en = jnp.sum(embedding ** 2, axis=1)
    idx2, minv2 = _dist_argmin(z.reshape(B, C, H * W), embedding,
                               en.reshape(_K, 1))
    tabt = jnp.transpose(embedding).reshape(-1)
    z_q = _make_sc_gather()(tabt, idx2.reshape(B, H * W)).reshape(B, C, H, W)
    indices = idx2.reshape(B, H, W)
    m = jnp.sum(minv2) / jnp.float32(z.size)
    loss = m + beta * m
    return (z_q, indices, loss)


# speedup vs baseline: 1.6316x; 1.6316x over previous
import functools

import jax
import jax.numpy as jnp
from jax import lax
from jax.experimental import pallas as pl
from jax.experimental.pallas import tpu as pltpu
from jax.experimental.pallas import tpu_sc as plsc

_K = 1024
_D = 256
_ROWS = 8192
_BLK = 1024

_NC = 2
_NS = 16
_NW = _NC * _NS
_ROWS_PER_W = _ROWS // _NW


_EXP = 0x4B000000


def _dist_argmin_body(z_ref, emb_ref, en_ref, idx_ref, minv_ref):
    zb = z_ref[0]
    emb = emb_ref[...]
    en = en_ref[...]
    rn = jnp.sum(zb * zb, axis=0, keepdims=True)
    p = lax.dot_general(emb, zb, (((1,), (0,)), ((), ())),
                        preferred_element_type=jnp.float32)
    d = (rn + en) - 2.0 * p
    minv = jnp.min(d, axis=0, keepdims=True)
    wio = lax.bitcast_convert_type(
        lax.broadcasted_iota(jnp.int32, d.shape, 0) + jnp.int32(_EXP),
        jnp.float32)
    packed = jnp.min(jnp.where(d == minv, wio, jnp.float32(3e38)),
                     axis=0, keepdims=True)
    idx = lax.bitcast_convert_type(packed, jnp.int32) - jnp.int32(_EXP)
    idx_ref[...] = idx[None]
    minv_ref[...] = minv[None]


_S = 1024

_dist_argmin = pl.pallas_call(
    _dist_argmin_body,
    grid=(8,),
    in_specs=[
        pl.BlockSpec((1, _D, _S), lambda i: (i, 0, 0)),
        pl.BlockSpec((_K, _D), lambda i: (0, 0)),
        pl.BlockSpec((_K, 1), lambda i: (0, 0)),
    ],
    out_specs=[
        pl.BlockSpec((1, 1, _S), lambda i: (i, 0, 0)),
        pl.BlockSpec((1, 1, _S), lambda i: (i, 0, 0)),
    ],
    out_shape=[
        jax.ShapeDtypeStruct((8, 1, _S), jnp.int32),
        jax.ShapeDtypeStruct((8, 1, _S), jnp.float32),
    ],
)


@functools.lru_cache(maxsize=1)
def _make_sc_gather():
    @functools.partial(
        pl.kernel,
        mesh=plsc.VectorSubcoreMesh(core_axis_name="c", subcore_axis_name="s"),
        out_type=jax.ShapeDtypeStruct((_ROWS, _D), jnp.float32),
        scratch_types=[
            pltpu.VMEM((_ROWS_PER_W,), jnp.int32),
            pltpu.VMEM((_ROWS_PER_W, _D), jnp.float32),
            pltpu.SemaphoreType.DMA,
        ],
    )
    def _sc_gather(table_hbm, idx_hbm, out_hbm, idx_v, rows_v, sem):
        wid = lax.axis_index("s") * _NC + lax.axis_index("c")
        base = wid * _ROWS_PER_W
        pltpu.sync_copy(idx_hbm.at[pl.ds(base, _ROWS_PER_W)], idx_v)
        pltpu.async_copy(table_hbm.at[idx_v], rows_v, sem).wait()
        pltpu.sync_copy(rows_v, out_hbm.at[pl.ds(base, _ROWS_PER_W)])

    return _sc_gather


def kernel(z, embedding):
    beta = 0.25
    B, C, H, W = z.shape
    en = jnp.sum(embedding ** 2, axis=1)
    idx2, minv2 = _dist_argmin(z.reshape(B, C, H * W), embedding,
                               en.reshape(_K, 1))
    zq_rows = _make_sc_gather()(embedding, idx2.reshape(-1))
    z_q = zq_rows.reshape(B, H, W, C).transpose(0, 3, 1, 2)
    indices = idx2.reshape(B, H, W)
    m = jnp.sum(minv2) / jnp.float32(z.size)
    loss = m + beta * m
    return (z_q, indices, loss)
